# Initial kernel scaffold; baseline (speedup 1.0000x reference)
import jax
import jax.numpy as jnp
from jax.experimental import pallas as pl


def kernel(x, codes):
    raise NotImplementedError("write your pallas kernel here")


# speedup vs baseline: 2.9012x; 2.9012x over previous
import jax
import jax.numpy as jnp
from jax.experimental import pallas as pl

_DISTANCE_THRESHOLD = 50.0
_NO_CODE_ID = -1
_BN = 512


def _nn_body(x_ref, codes_ref, out_ref):
    x = x_ref[...]
    codes = codes_ref[...]
    cross = jax.lax.dot_general(
        x, codes, (((1,), (1,)), ((), ())),
        preferred_element_type=jnp.float32,
    )
    x_sq = jnp.sum(x * x, axis=1, keepdims=True)
    c_sq = jnp.sum(codes * codes, axis=1)
    dist = (x_sq + c_sq[None, :]) - 2.0 * cross
    idx = jnp.argmin(dist, axis=1).astype(jnp.int32)
    min_d = jnp.min(dist, axis=1)
    out_ref[0, 0, :] = jnp.where(min_d <= _DISTANCE_THRESHOLD,
                                 idx, _NO_CODE_ID)


def kernel(x, codes):
    b, n, d = x.shape
    k = codes.shape[0]
    tokens = b * n
    xf = x.reshape(tokens, d)
    num_blocks = tokens // _BN
    out = pl.pallas_call(
        _nn_body,
        grid=(num_blocks,),
        in_specs=[
            pl.BlockSpec((_BN, d), lambda i: (i, 0)),
            pl.BlockSpec((k, d), lambda i: (0, 0)),
        ],
        out_specs=pl.BlockSpec((1, 1, _BN), lambda i: (i, 0, 0)),
        out_shape=jax.ShapeDtypeStruct((num_blocks, 1, _BN), jnp.int32),
    )(xf, codes)
    return out.reshape(b, n)


# speedup vs baseline: 2.9677x; 1.0229x over previous
import jax
import jax.numpy as jnp
from jax.experimental import pallas as pl
from jax.experimental.pallas import tpu as pltpu

_DISTANCE_THRESHOLD = 50.0
_NO_CODE_ID = -1
_BN = 512


def _nn_body(x_ref, codes_ref, out_ref, csq_ref, cm2_ref):
    i = pl.program_id(0)

    @pl.when(i == 0)
    def _prep():
        c = codes_ref[...]
        csq_ref[...] = jnp.sum(c * c, axis=1)[None, :]
        cm2_ref[...] = c * (-2.0)

    x = x_ref[...]
    cross2 = jax.lax.dot_general(
        x, cm2_ref[...], (((1,), (1,)), ((), ())),
        preferred_element_type=jnp.float32,
    )
    x_sq = jnp.sum(x * x, axis=1, keepdims=True)
    dist = (x_sq + csq_ref[...]) + cross2
    idx = jnp.argmin(dist, axis=1).astype(jnp.int32)
    min_d = jnp.min(dist, axis=1)
    out_ref[0, 0, :] = jnp.where(min_d <= _DISTANCE_THRESHOLD,
                                 idx, _NO_CODE_ID)


def kernel(x, codes):
    b, n, d = x.shape
    k = codes.shape[0]
    tokens = b * n
    xf = x.reshape(tokens, d)
    num_blocks = tokens // _BN
    out = pl.pallas_call(
        _nn_body,
        grid=(num_blocks,),
        in_specs=[
            pl.BlockSpec((_BN, d), lambda i: (i, 0)),
            pl.BlockSpec((k, d), lambda i: (0, 0)),
        ],
        out_specs=pl.BlockSpec((1, 1, _BN), lambda i: (i, 0, 0)),
        out_shape=jax.ShapeDtypeStruct((num_blocks, 1, _BN), jnp.int32),
        scratch_shapes=[
            pltpu.VMEM((1, k), jnp.float32),
            pltpu.VMEM((k, d), jnp.float32),
        ],
    )(xf, codes)
    return out.reshape(b, n)


# speedup vs baseline: 3.3471x; 1.1279x over previous
import jax
import jax.numpy as jnp
from jax.experimental import pallas as pl
from jax.experimental.pallas import tpu as pltpu

_DISTANCE_THRESHOLD = 50.0
_NO_CODE_ID = -1
_BN = 512


def _nn_body(x_ref, codes_ref, out_ref, csq_ref, cm2_ref):
    i = pl.program_id(0)

    @pl.when(i == 0)
    def _prep():
        c = codes_ref[...]
        csq_ref[...] = jnp.sum(c * c, axis=1)[None, :]
        cm2_ref[...] = c * (-2.0)

    x = x_ref[...]
    cross2 = jax.lax.dot_general(
        x, cm2_ref[...], (((1,), (1,)), ((), ())),
        preferred_element_type=jnp.float32,
    )
    x_sq = jnp.sum(x * x, axis=1, keepdims=True)
    csq = csq_ref[...]

    bn = x.shape[0]
    k = csq.shape[1]
    c = 128
    minv = (x_sq + csq[:, 0:c]) + cross2[:, 0:c]
    mini = jnp.zeros((bn, c), jnp.int32)
    for j in range(1, k // c):
        d = (x_sq + csq[:, j * c:(j + 1) * c]) + cross2[:, j * c:(j + 1) * c]
        better = d < minv
        minv = jnp.where(better, d, minv)
        mini = jnp.where(better, j, mini)
    lane = jax.lax.broadcasted_iota(jnp.int32, (bn, c), 1)
    gmin = jnp.min(minv, axis=1)
    k_arr = mini * c + lane
    k_cand = jnp.where(minv == gmin[:, None], k_arr, k)
    idx = jnp.min(k_cand, axis=1)
    out_ref[0, 0, :] = jnp.where(gmin <= _DISTANCE_THRESHOLD,
                                 idx, _NO_CODE_ID)


def kernel(x, codes):
    b, n, d = x.shape
    k = codes.shape[0]
    tokens = b * n
    xf = x.reshape(tokens, d)
    num_blocks = tokens // _BN
    out = pl.pallas_call(
        _nn_body,
        grid=(num_blocks,),
        in_specs=[
            pl.BlockSpec((_BN, d), lambda i: (i, 0)),
            pl.BlockSpec((k, d), lambda i: (0, 0)),
        ],
        out_specs=pl.BlockSpec((1, 1, _BN), lambda i: (i, 0, 0)),
        out_shape=jax.ShapeDtypeStruct((num_blocks, 1, _BN), jnp.int32),
        scratch_shapes=[
            pltpu.VMEM((1, k), jnp.float32),
            pltpu.VMEM((k, d), jnp.float32),
        ],
    )(xf, codes)
    return out.reshape(b, n)


# speedup vs baseline: 3.3719x; 1.0074x over previous
import jax
import jax.numpy as jnp
from jax.experimental import pallas as pl
from jax.experimental.pallas import tpu as pltpu

_DISTANCE_THRESHOLD = 50.0
_NO_CODE_ID = -1
_BN = 1152


def _nn_body(x_ref, codes_ref, out_ref, csq_ref, cm2_ref):
    i = pl.program_id(0)

    @pl.when(i == 0)
    def _prep():
        c = codes_ref[...]
        csq_ref[...] = jnp.sum(c * c, axis=1)[None, :]
        cm2_ref[...] = c * (-2.0)

    x = x_ref[...]
    cross2 = jax.lax.dot_general(
        x, cm2_ref[...], (((1,), (1,)), ((), ())),
        preferred_element_type=jnp.float32,
    )
    x_sq = jnp.sum(x * x, axis=1, keepdims=True)
    csq = csq_ref[...]

    bn = x.shape[0]
    k = csq.shape[1]
    c = 128
    minv = (x_sq + csq[:, 0:c]) + cross2[:, 0:c]
    mini = jnp.zeros((bn, c), jnp.int32)
    for j in range(1, k // c):
        d = (x_sq + csq[:, j * c:(j + 1) * c]) + cross2[:, j * c:(j + 1) * c]
        better = d < minv
        minv = jnp.where(better, d, minv)
        mini = jnp.where(better, j, mini)
    lane = jax.lax.broadcasted_iota(jnp.int32, (bn, c), 1)
    gmin = jnp.min(minv, axis=1)
    k_arr = mini * c + lane
    k_cand = jnp.where(minv == gmin[:, None], k_arr, k)
    idx = jnp.min(k_cand, axis=1)
    out_ref[0, 0, :] = jnp.where(gmin <= _DISTANCE_THRESHOLD,
                                 idx, _NO_CODE_ID)


def kernel(x, codes):
    b, n, d = x.shape
    k = codes.shape[0]
    tokens = b * n
    xf = x.reshape(tokens, d)
    num_blocks = tokens // _BN
    out = pl.pallas_call(
        _nn_body,
        grid=(num_blocks,),
        in_specs=[
            pl.BlockSpec((_BN, d), lambda i: (i, 0)),
            pl.BlockSpec((k, d), lambda i: (0, 0)),
        ],
        out_specs=pl.BlockSpec((1, 1, _BN), lambda i: (i, 0, 0)),
        out_shape=jax.ShapeDtypeStruct((num_blocks, 1, _BN), jnp.int32),
        scratch_shapes=[
            pltpu.VMEM((1, k), jnp.float32),
            pltpu.VMEM((k, d), jnp.float32),
        ],
    )(xf, codes)
    return out.reshape(b, n)


# speedup vs baseline: 4.0158x; 1.1909x over previous
import jax
import jax.numpy as jnp
from jax.experimental import pallas as pl

_DISTANCE_THRESHOLD = 50.0
_NO_CODE_ID = -1
_BT = 512
_C = 128


def _nn_body(x_ref, codes_ref, out_ref):
    codes = codes_ref[...]
    k = codes.shape[0]
    cm2 = codes * (-2.0)
    csq = jnp.sum(codes * codes, axis=1)[None, :]
    tokens = x_ref.shape[0]
    for t in range(tokens // _BT):
        xt = x_ref[pl.ds(t * _BT, _BT), :]
        x_sq = jnp.sum(xt * xt, axis=1, keepdims=True)
        cross2 = jax.lax.dot_general(
            xt, cm2, (((1,), (1,)), ((), ())),
            preferred_element_type=jnp.float32,
        )
        minv = (x_sq + csq[:, 0:_C]) + cross2[:, 0:_C]
        mini = jnp.zeros((_BT, _C), jnp.int32)
        for j in range(1, k // _C):
            d = (x_sq + csq[:, j * _C:(j + 1) * _C]) + cross2[:, j * _C:(j + 1) * _C]
            better = d < minv
            minv = jnp.where(better, d, minv)
            mini = jnp.where(better, j, mini)
        lane = jax.lax.broadcasted_iota(jnp.int32, (_BT, _C), 1)
        gmin = jnp.min(minv, axis=1)
        k_arr = mini * _C + lane
        k_cand = jnp.where(minv == gmin[:, None], k_arr, k)
        idx = jnp.min(k_cand, axis=1)
        out_ref[0, pl.ds(t * _BT, _BT)] = jnp.where(
            gmin <= _DISTANCE_THRESHOLD, idx, _NO_CODE_ID)


def kernel(x, codes):
    b, n, d = x.shape
    tokens = b * n
    xf = x.reshape(tokens, d)
    out = pl.pallas_call(
        _nn_body,
        out_shape=jax.ShapeDtypeStruct((1, tokens), jnp.int32),
    )(xf, codes)
    return out.reshape(b, n)
